# baseline (speedup 1.0000x reference)
import functools

import jax
import jax.numpy as jnp
from jax import lax
from jax.experimental import pallas as pl
from jax.experimental.pallas import tpu as pltpu
from jax.experimental.pallas import tpu_sc as plsc

N_NODES = 10000
N_EDGES = 160000
D_IN = 256
D_OUT = 256
DH = D_IN // 2

NC = 2
NS = 16
L = 16
CHUNK = 128
NCHUNK = 80
E_PER_TILE = NCHUNK * CHUNK
E_PAD = NS * E_PER_TILE

NPASS = 2
H_ROWS = 10240
PROWS = H_ROWS // NPASS
TRASH = PROWS
A_ROWS = PROWS + 128
ZROWS = A_ROWS // NS
OROWS = PROWS // NS

DBITS = 14
DMASK = (1 << DBITS) - 1
CROWS = NCHUNK + 2
CCAP = CROWS * CHUNK


def _sc_agg_build():
    mesh = plsc.VectorSubcoreMesh(core_axis_name="c", subcore_axis_name="s")

    @functools.partial(
        pl.kernel,
        mesh=mesh,
        compiler_params=pltpu.CompilerParams(needs_layout_passes=False),
        out_type=jax.ShapeDtypeStruct((NC, H_ROWS, DH), jnp.float32),
        scratch_types=[
            pltpu.VMEM((E_PER_TILE,), jnp.int32),
            pltpu.VMEM((CCAP,), jnp.int32),
            pltpu.VMEM((CCAP,), jnp.int32),
            pltpu.VMEM((CCAP,), jnp.int32),
            pltpu.VMEM((CROWS, CHUNK), jnp.int32),
            pltpu.VMEM((3 * L,), jnp.int32),
            pltpu.SMEM((8,), jnp.int32),
            pltpu.VMEM((CHUNK, DH), jnp.float32),
            pltpu.VMEM((CHUNK, DH), jnp.float32),
            pltpu.VMEM_SHARED((A_ROWS, DH), jnp.float32),
            pltpu.SemaphoreType.DMA,
            pltpu.SemaphoreType.DMA,
        ],
    )
    def sc_agg(x2, combo_hbm, nch_hbm, zeros, out,
               combo, cc0, cc1, srcf, dst2d, pbuf, nch_sm, rows0, rows1, h_sh,
               sem0, sem1):
        c = lax.axis_index("c")
        s = lax.axis_index("s")

        pltpu.sync_copy(combo_hbm.at[s], combo)

        def pbody(g, _):
            cc0[pl.ds(g * L, L)] = jnp.full((L,), TRASH, jnp.int32)
            cc1[pl.ds(g * L, L)] = jnp.full((L,), PROWS + TRASH, jnp.int32)
            return _

        lax.fori_loop(0, CCAP // L, pbody, None)

        pbuf[pl.ds(0, L)] = jnp.zeros((L,), jnp.int32)
        pbuf[pl.ds(2 * L, L)] = jnp.zeros((L,), jnp.int32)
        li = lax.iota(jnp.int32, L)
        zv = jnp.zeros((L,), jnp.int32)

        def cbody(g, offs):
            off0, off1 = offs
            v = combo[pl.ds(g * L, L)]
            m0 = (v & DMASK) < PROWS
            csum = plsc.cumsum(m0.astype(jnp.int32))
            pos0 = off0 + csum - 1
            pos1 = off1 + li - csum
            plsc.store_scatter(cc0, [jnp.where(m0, pos0, zv)], v, mask=m0)
            plsc.store_scatter(cc1, [jnp.where(m0, zv, pos1)], v,
                               mask=jnp.logical_not(m0))
            cnt = csum[L - 1]
            return (off0 + cnt, off1 + (L - cnt))

        n0, n1 = lax.fori_loop(0, E_PER_TILE // L, cbody, (0, 0))

        for p in range(NPASS):
            cc = cc0 if p == 0 else cc1
            ncnt = n0 if p == 0 else n1
            nch = (ncnt + CHUNK - 1) // CHUNK
            plsc.subcore_barrier()
            pltpu.sync_copy(zeros, h_sh.at[pl.ds(s * ZROWS, ZROWS)])

            def ubody(g, _):
                v = cc[pl.ds(g * L, L)]
                srcf[pl.ds(g * L, L)] = (v >> DBITS) + c * N_NODES
                d = (v & DMASK) - (p * PROWS)
                d = jnp.where((d >= 0) & (d < PROWS), d, TRASH)
                dst2d[g // (CHUNK // L), pl.ds((g % (CHUNK // L)) * L, L)] = d
                return _

            lax.fori_loop(0, CCAP // L, ubody, None)
            plsc.subcore_barrier()

            for k, (buf, sem) in enumerate(((rows0, sem0), (rows1, sem1))):
                @pl.when(k < nch)
                def _prime():
                    pltpu.async_copy(
                        x2.at[srcf.at[pl.ds(k * CHUNK, CHUNK)]], buf, sem)

            def body(j2, _):
                for k, (buf, sem) in enumerate(((rows0, sem0), (rows1, sem1))):
                    j = 2 * j2 + k

                    @pl.when(j < nch)
                    def _work():
                        pltpu.make_async_copy(
                            x2.at[srcf.at[pl.ds(j * CHUNK, CHUNK)]],
                            buf, sem).wait()
                        pltpu.sync_copy(buf, h_sh.at[dst2d.at[j]], add=True)
                        nxt = j + 2

                        @pl.when(nxt < nch)
                        def _start():
                            pltpu.async_copy(
                                x2.at[srcf.at[pl.ds(nxt * CHUNK, CHUNK)]],
                                buf, sem)
                return _

            lax.fori_loop(0, CROWS // 2, body, None)
            plsc.subcore_barrier()

            pltpu.sync_copy(h_sh.at[pl.ds(s * OROWS, OROWS)],
                            out.at[c, pl.ds(p * PROWS + s * OROWS, OROWS)])

    return sc_agg


_sc_agg = _sc_agg_build()


BM = 1000


def _mm_body(h2_ref, wt_ref, b_ref, o_ref):
    acc = b_ref[...]
    for q in range(NC):
        acc = acc + jnp.dot(h2_ref[q], wt_ref[q * DH:(q + 1) * DH],
                            preferred_element_type=jnp.float32)
    o_ref[...] = jnp.maximum(acc, 0.0)


def _tc_linear(h2, w_t, b2):
    return pl.pallas_call(
        _mm_body,
        grid=(N_NODES // BM,),
        in_specs=[
            pl.BlockSpec((NC, BM, DH), lambda i: (0, i, 0)),
            pl.BlockSpec((D_IN, D_OUT), lambda i: (0, 0)),
            pl.BlockSpec((1, D_OUT), lambda i: (0, 0)),
        ],
        out_specs=pl.BlockSpec((BM, D_OUT), lambda i: (i, 0)),
        out_shape=jax.ShapeDtypeStruct((N_NODES, D_OUT), jnp.float32),
    )(h2, w_t, b2)


def kernel(x, edge_index, W, b):
    src = edge_index[0].astype(jnp.int32)
    dst = edge_index[1].astype(jnp.int32)
    pad = E_PAD - N_EDGES
    src_p = jnp.concatenate([src, jnp.zeros((pad,), jnp.int32)])
    dst_p = jnp.concatenate([dst, jnp.full((pad,), PROWS + TRASH, jnp.int32)])
    combo = ((src_p << DBITS) | dst_p).reshape(NS, E_PER_TILE)
    n0s = jnp.sum((dst_p < PROWS).reshape(NS, E_PER_TILE), axis=1)
    nch0 = (n0s + CHUNK - 1) // CHUNK
    nch1 = (E_PER_TILE - n0s + CHUNK - 1) // CHUNK
    nch = jnp.stack([nch0, nch1] + [nch0] * 6, axis=1).astype(jnp.int32)
    nch = nch.reshape(NS, 1, 8)
    x2 = jnp.concatenate([x[:, :DH], x[:, DH:]], axis=0)
    zeros = jnp.zeros((ZROWS, DH), jnp.float32)

    h2 = _sc_agg(x2, combo, nch, zeros)

    w_t = jnp.transpose(W)
    b2 = b.reshape(1, D_OUT)
    return _tc_linear(h2, w_t, b2)
